# Initial kernel scaffold; baseline (speedup 1.0000x reference)
import functools

import jax
import jax.numpy as jnp
from jax import lax
from jax.experimental import pallas as pl
from jax.experimental.pallas import tpu as pltpu
from jax.experimental.pallas import tpu_sc as plsc

N_NODES = 10000
N_EDGES = 320000
NODE_NF = 11
EDGE_NF = 4
H_NF = 128
EMB_NF = 4

NC = 2
NS = 16
NW = NC * NS

CH = 128
EPW = 10240
NCH = EPW // CH
E_PAD = EPW * NW
N_PAD = 10016
RPT = N_PAD // NS

_F32 = jnp.float32


def _sc_mesh():
    return plsc.VectorSubcoreMesh(
        core_axis_name="c", subcore_axis_name="s", num_cores=NC, num_subcores=NS
    )


def _gather_call(nf16, row_pad, col_pad):
    @functools.partial(
        pl.kernel,
        out_type=(
            jax.ShapeDtypeStruct((E_PAD, 16), _F32),
            jax.ShapeDtypeStruct((E_PAD, 16), _F32),
        ),
        mesh=_sc_mesh(),
        scratch_types=[
            pltpu.VMEM((CH,), jnp.int32),
            pltpu.VMEM((CH,), jnp.int32),
            pltpu.VMEM((CH, 16), _F32),
            pltpu.VMEM((CH, 16), _F32),
            pltpu.SemaphoreType.DMA,
            pltpu.SemaphoreType.DMA,
        ],
    )
    def k(nf_hbm, row_hbm, col_hbm, gsrc_hbm, gdst_hbm,
          ridx_v, cidx_v, srow_v, drow_v, sem1, sem2):
        wid = lax.axis_index("c") * NS + lax.axis_index("s")

        def body(t, carry):
            base = pl.multiple_of(wid * EPW + t * CH, CH)
            pltpu.sync_copy(row_hbm.at[pl.ds(base, CH)], ridx_v)
            pltpu.sync_copy(col_hbm.at[pl.ds(base, CH)], cidx_v)
            cp1 = pltpu.async_copy(nf_hbm.at[ridx_v], srow_v, sem1)
            cp2 = pltpu.async_copy(nf_hbm.at[cidx_v], drow_v, sem2)
            cp1.wait()
            cp2.wait()
            pltpu.sync_copy(srow_v, gsrc_hbm.at[pl.ds(base, CH)])
            pltpu.sync_copy(drow_v, gdst_hbm.at[pl.ds(base, CH)])
            return carry

        lax.fori_loop(0, NCH, body, 0, unroll=False)

    return k(nf16, row_pad, col_pad)


def _scatter_call(ef, row_pad, zeros_big):
    @functools.partial(
        pl.kernel,
        out_type=(
            jax.ShapeDtypeStruct((N_PAD, H_NF), _F32),
            jax.ShapeDtypeStruct((N_PAD, H_NF), _F32),
        ),
        mesh=_sc_mesh(),
        scratch_types=[
            pltpu.VMEM((CH,), jnp.int32),
            pltpu.VMEM((CH, H_NF), _F32),
            pltpu.VMEM_SHARED((N_PAD, H_NF), _F32),
            pltpu.SemaphoreType.DMA,
        ],
    )
    def k(ef_hbm, row_hbm, z_hbm, p0_hbm, p1_hbm, idx_v, rows_v, agg_sh, sem):
        c = lax.axis_index("c")
        s = lax.axis_index("s")
        wid = c * NS + s
        rslice = pl.ds(s * RPT, RPT)
        pltpu.sync_copy(z_hbm.at[rslice], agg_sh.at[rslice])
        plsc.subcore_barrier()

        def body(t, carry):
            base = pl.multiple_of(wid * EPW + t * CH, CH)
            pltpu.sync_copy(row_hbm.at[pl.ds(base, CH)], idx_v)
            pltpu.sync_copy(ef_hbm.at[pl.ds(base, CH)], rows_v)
            pltpu.sync_copy(rows_v, agg_sh.at[idx_v], add=True)
            return carry

        lax.fori_loop(0, NCH, body, 0, unroll=False)
        plsc.subcore_barrier()

        @pl.when(c == 0)
        def _():
            pltpu.sync_copy(agg_sh.at[rslice], p0_hbm.at[rslice])

        @pl.when(c == 1)
        def _():
            pltpu.sync_copy(agg_sh.at[rslice], p1_hbm.at[rslice])

    return k(ef, row_pad, zeros_big)


BE = 2048


def _edge_mlp_kernel(gsrc, gdst, ea, w1s, w1d, w1e, b1, w2, b2, out):
    h = jnp.dot(gsrc[...], w1s[...], preferred_element_type=_F32)
    h = h + jnp.dot(gdst[...], w1d[...], preferred_element_type=_F32)
    h = h + jnp.dot(ea[...], w1e[...], preferred_element_type=_F32)
    h = jnp.maximum(h + b1[...], 0.0)
    h = jnp.dot(h, w2[...], preferred_element_type=_F32) + b2[...]
    out[...] = jnp.maximum(h, 0.0)


def _edge_mlp_call(gsrc, gdst, ea8, w1s, w1d, w1e, b1, w2, b2):
    grid = (E_PAD // BE,)
    bcast = lambda shape: pl.BlockSpec(shape, lambda i: (0, 0))
    return pl.pallas_call(
        _edge_mlp_kernel,
        grid=grid,
        in_specs=[
            pl.BlockSpec((BE, 16), lambda i: (i, 0)),
            pl.BlockSpec((BE, 16), lambda i: (i, 0)),
            pl.BlockSpec((BE, 8), lambda i: (i, 0)),
            bcast((16, H_NF)),
            bcast((16, H_NF)),
            bcast((8, H_NF)),
            bcast((1, H_NF)),
            bcast((H_NF, H_NF)),
            bcast((1, H_NF)),
        ],
        out_specs=pl.BlockSpec((BE, H_NF), lambda i: (i, 0)),
        out_shape=jax.ShapeDtypeStruct((E_PAD, H_NF), _F32),
    )(gsrc, gdst, ea8, w1s, w1d, w1e, b1, w2, b2)


BN = 1024


def _node_mlp_kernel(nf, p0, p1, w1n, w1a, b1, w2, b2, fw, fb, out):
    agg = p0[...] + p1[...]
    h = jnp.dot(nf[...], w1n[...], preferred_element_type=_F32)
    h = h + jnp.dot(agg, w1a[...], preferred_element_type=_F32)
    h = jnp.maximum(h + b1[...], 0.0)
    h = jnp.dot(h, w2[...], preferred_element_type=_F32) + b2[...]
    out[...] = jnp.dot(h, fw[...], preferred_element_type=_F32) + fb[...]


def _node_mlp_call(nf16, p0, p1, w1n, w1a, b1, w2, b2, fw8, fb8):
    grid = (pl.cdiv(N_NODES, BN),)
    bcast = lambda shape: pl.BlockSpec(shape, lambda i: (0, 0))
    return pl.pallas_call(
        _node_mlp_kernel,
        grid=grid,
        in_specs=[
            pl.BlockSpec((BN, 16), lambda i: (i, 0)),
            pl.BlockSpec((BN, H_NF), lambda i: (i, 0)),
            pl.BlockSpec((BN, H_NF), lambda i: (i, 0)),
            bcast((16, H_NF)),
            bcast((H_NF, H_NF)),
            bcast((1, H_NF)),
            bcast((H_NF, H_NF)),
            bcast((1, H_NF)),
            bcast((H_NF, 8)),
            bcast((1, 8)),
        ],
        out_specs=pl.BlockSpec((BN, 8), lambda i: (i, 0)),
        out_shape=jax.ShapeDtypeStruct((N_NODES, 8), _F32),
    )(nf16, p0, p1, w1n, w1a, b1, w2, b2, fw8, fb8)


def kernel(node_feats, edge_index, edge_attr,
           eW1, eb1, eW2, eb2, nW1, nb1, nW2, nb2, fW, fb):
    row = edge_index[0]
    col = edge_index[1]
    pad_idx = jnp.full((E_PAD - N_EDGES,), N_NODES, jnp.int32)
    row_pad = jnp.concatenate([row, pad_idx])
    col_pad = jnp.concatenate([col, pad_idx])

    nf16 = jnp.zeros((N_PAD, 16), _F32).at[:N_NODES, :NODE_NF].set(node_feats)
    ea8 = jnp.zeros((E_PAD, 8), _F32).at[:N_EDGES, :EDGE_NF].set(edge_attr)

    w1s = jnp.zeros((16, H_NF), _F32).at[:NODE_NF].set(eW1[:NODE_NF])
    w1d = jnp.zeros((16, H_NF), _F32).at[:NODE_NF].set(eW1[NODE_NF:2 * NODE_NF])
    w1e = jnp.zeros((8, H_NF), _F32).at[:EDGE_NF].set(eW1[2 * NODE_NF:])
    e_b1 = eb1.reshape(1, H_NF)
    e_b2 = eb2.reshape(1, H_NF)

    w1n = jnp.zeros((16, H_NF), _F32).at[:NODE_NF].set(nW1[:NODE_NF])
    w1a = nW1[NODE_NF:]
    n_b1 = nb1.reshape(1, H_NF)
    n_b2 = nb2.reshape(1, H_NF)
    fw8 = jnp.zeros((H_NF, 8), _F32).at[:, :EMB_NF].set(fW)
    fb8 = jnp.zeros((1, 8), _F32).at[0, :EMB_NF].set(fb)

    gsrc, gdst = _gather_call(nf16, row_pad, col_pad)
    ef = _edge_mlp_call(gsrc, gdst, ea8, w1s, w1d, w1e, e_b1, eW2, e_b2)
    zeros_big = jnp.zeros((N_PAD, H_NF), _F32)
    p0, p1 = _scatter_call(ef, row_pad, zeros_big)
    out8 = _node_mlp_call(nf16, p0, p1, w1n, w1a, n_b1, nW2, n_b2, fw8, fb8)
    return out8[:, :EMB_NF]


# speedup vs baseline: 2.4108x; 2.4108x over previous
import functools

import jax
import jax.numpy as jnp
from jax import lax
from jax.experimental import pallas as pl
from jax.experimental.pallas import tpu as pltpu
from jax.experimental.pallas import tpu_sc as plsc

N_NODES = 10000
N_EDGES = 320000
NODE_NF = 11
EDGE_NF = 4
H_NF = 128
EMB_NF = 4

NC = 2
NS = 16
NW = NC * NS

CH = 128
EPW = 10240
NCH = EPW // CH
E_PAD = EPW * NW
N_PAD = 10112
RPT = N_PAD // NS

_F32 = jnp.float32


def _sc_mesh():
    return plsc.VectorSubcoreMesh(
        core_axis_name="c", subcore_axis_name="s", num_cores=NC, num_subcores=NS
    )


def _gather_call(nf16, row_pad, col_pad):
    @functools.partial(
        pl.kernel,
        out_type=(
            jax.ShapeDtypeStruct((E_PAD, 16), _F32),
            jax.ShapeDtypeStruct((E_PAD, 16), _F32),
        ),
        mesh=_sc_mesh(),
        scratch_types=[
            pltpu.VMEM((CH,), jnp.int32),
            pltpu.VMEM((CH,), jnp.int32),
            pltpu.VMEM((CH, 16), _F32),
            pltpu.VMEM((CH, 16), _F32),
            pltpu.SemaphoreType.DMA,
            pltpu.SemaphoreType.DMA,
        ],
        compiler_params=pltpu.CompilerParams(use_tc_tiling_on_sc=False),
    )
    def k(nf_hbm, row_hbm, col_hbm, gsrc_hbm, gdst_hbm,
          ridx_v, cidx_v, srow_v, drow_v, sem1, sem2):
        wid = lax.axis_index("c") * NS + lax.axis_index("s")

        def body(t, carry):
            base = pl.multiple_of(wid * EPW + t * CH, CH)
            pltpu.sync_copy(row_hbm.at[pl.ds(base, CH)], ridx_v)
            pltpu.sync_copy(col_hbm.at[pl.ds(base, CH)], cidx_v)
            cp1 = pltpu.async_copy(nf_hbm.at[ridx_v], srow_v, sem1)
            cp2 = pltpu.async_copy(nf_hbm.at[cidx_v], drow_v, sem2)
            cp1.wait()
            cp2.wait()
            pltpu.sync_copy(srow_v, gsrc_hbm.at[pl.ds(base, CH)])
            pltpu.sync_copy(drow_v, gdst_hbm.at[pl.ds(base, CH)])
            return carry

        lax.fori_loop(0, NCH, body, 0, unroll=False)

    return k(nf16, row_pad, col_pad)


def _scatter_call(ef, row_pad, zeros_big):
    @functools.partial(
        pl.kernel,
        out_type=(
            jax.ShapeDtypeStruct((N_PAD, H_NF), _F32),
            jax.ShapeDtypeStruct((N_PAD, H_NF), _F32),
        ),
        mesh=_sc_mesh(),
        scratch_types=[
            pltpu.VMEM((CH,), jnp.int32),
            pltpu.VMEM((CH, H_NF), _F32),
            pltpu.VMEM_SHARED((N_PAD, H_NF), _F32),
            pltpu.SemaphoreType.DMA,
        ],
    )
    def k(ef_hbm, row_hbm, z_hbm, p0_hbm, p1_hbm, idx_v, rows_v, agg_sh, sem):
        c = lax.axis_index("c")
        s = lax.axis_index("s")
        wid = c * NS + s
        rslice = pl.ds(s * RPT, RPT)
        pltpu.sync_copy(z_hbm.at[rslice], agg_sh.at[rslice])
        plsc.subcore_barrier()

        def body(t, carry):
            base = pl.multiple_of(wid * EPW + t * CH, CH)
            pltpu.sync_copy(row_hbm.at[pl.ds(base, CH)], idx_v)
            pltpu.sync_copy(ef_hbm.at[pl.ds(base, CH)], rows_v)
            pltpu.sync_copy(rows_v, agg_sh.at[idx_v], add=True)
            return carry

        lax.fori_loop(0, NCH, body, 0, unroll=False)
        plsc.subcore_barrier()

        @pl.when(c == 0)
        def _():
            pltpu.sync_copy(agg_sh.at[rslice], p0_hbm.at[rslice])

        @pl.when(c == 1)
        def _():
            pltpu.sync_copy(agg_sh.at[rslice], p1_hbm.at[rslice])

    return k(ef, row_pad, zeros_big)


BE = 2048


def _edge_mlp_kernel(gsrc, gdst, ea, w1s, w1d, w1e, b1, w2, b2, out):
    h = jnp.dot(gsrc[...], w1s[...], preferred_element_type=_F32)
    h = h + jnp.dot(gdst[...], w1d[...], preferred_element_type=_F32)
    h = h + jnp.dot(ea[...], w1e[...], preferred_element_type=_F32)
    h = jnp.maximum(h + b1[...], 0.0)
    h = jnp.dot(h, w2[...], preferred_element_type=_F32) + b2[...]
    out[...] = jnp.maximum(h, 0.0)


def _edge_mlp_call(gsrc, gdst, ea8, w1s, w1d, w1e, b1, w2, b2):
    grid = (E_PAD // BE,)
    bcast = lambda shape: pl.BlockSpec(shape, lambda i: (0, 0))
    return pl.pallas_call(
        _edge_mlp_kernel,
        grid=grid,
        in_specs=[
            pl.BlockSpec((BE, 16), lambda i: (i, 0)),
            pl.BlockSpec((BE, 16), lambda i: (i, 0)),
            pl.BlockSpec((BE, 8), lambda i: (i, 0)),
            bcast((16, H_NF)),
            bcast((16, H_NF)),
            bcast((8, H_NF)),
            bcast((1, H_NF)),
            bcast((H_NF, H_NF)),
            bcast((1, H_NF)),
        ],
        out_specs=pl.BlockSpec((BE, H_NF), lambda i: (i, 0)),
        out_shape=jax.ShapeDtypeStruct((E_PAD, H_NF), _F32),
    )(gsrc, gdst, ea8, w1s, w1d, w1e, b1, w2, b2)


BN = 1024


def _node_mlp_kernel(nf, p0, p1, w1n, w1a, b1, w2, b2, fw, fb, out):
    agg = p0[...] + p1[...]
    h = jnp.dot(nf[...], w1n[...], preferred_element_type=_F32)
    h = h + jnp.dot(agg, w1a[...], preferred_element_type=_F32)
    h = jnp.maximum(h + b1[...], 0.0)
    h = jnp.dot(h, w2[...], preferred_element_type=_F32) + b2[...]
    out[...] = jnp.dot(h, fw[...], preferred_element_type=_F32) + fb[...]


def _node_mlp_call(nf16, p0, p1, w1n, w1a, b1, w2, b2, fw8, fb8):
    grid = (pl.cdiv(N_NODES, BN),)
    bcast = lambda shape: pl.BlockSpec(shape, lambda i: (0, 0))
    return pl.pallas_call(
        _node_mlp_kernel,
        grid=grid,
        in_specs=[
            pl.BlockSpec((BN, 16), lambda i: (i, 0)),
            pl.BlockSpec((BN, H_NF), lambda i: (i, 0)),
            pl.BlockSpec((BN, H_NF), lambda i: (i, 0)),
            bcast((16, H_NF)),
            bcast((H_NF, H_NF)),
            bcast((1, H_NF)),
            bcast((H_NF, H_NF)),
            bcast((1, H_NF)),
            bcast((H_NF, 8)),
            bcast((1, 8)),
        ],
        out_specs=pl.BlockSpec((BN, 8), lambda i: (i, 0)),
        out_shape=jax.ShapeDtypeStruct((N_NODES, 8), _F32),
    )(nf16, p0, p1, w1n, w1a, b1, w2, b2, fw8, fb8)


def kernel(node_feats, edge_index, edge_attr,
           eW1, eb1, eW2, eb2, nW1, nb1, nW2, nb2, fW, fb):
    row = edge_index[0]
    col = edge_index[1]
    pad_idx = jnp.full((E_PAD - N_EDGES,), N_NODES, jnp.int32)
    row_pad = jnp.concatenate([row, pad_idx])
    col_pad = jnp.concatenate([col, pad_idx])

    nf16 = jnp.zeros((N_PAD, 16), _F32).at[:N_NODES, :NODE_NF].set(node_feats)
    ea8 = jnp.zeros((E_PAD, 8), _F32).at[:N_EDGES, :EDGE_NF].set(edge_attr)

    w1s = jnp.zeros((16, H_NF), _F32).at[:NODE_NF].set(eW1[:NODE_NF])
    w1d = jnp.zeros((16, H_NF), _F32).at[:NODE_NF].set(eW1[NODE_NF:2 * NODE_NF])
    w1e = jnp.zeros((8, H_NF), _F32).at[:EDGE_NF].set(eW1[2 * NODE_NF:])
    e_b1 = eb1.reshape(1, H_NF)
    e_b2 = eb2.reshape(1, H_NF)

    w1n = jnp.zeros((16, H_NF), _F32).at[:NODE_NF].set(nW1[:NODE_NF])
    w1a = nW1[NODE_NF:]
    n_b1 = nb1.reshape(1, H_NF)
    n_b2 = nb2.reshape(1, H_NF)
    fw8 = jnp.zeros((H_NF, 8), _F32).at[:, :EMB_NF].set(fW)
    fb8 = jnp.zeros((1, 8), _F32).at[0, :EMB_NF].set(fb)

    gsrc, gdst = _gather_call(nf16, row_pad, col_pad)
    ef = _edge_mlp_call(gsrc, gdst, ea8, w1s, w1d, w1e, e_b1, eW2, e_b2)
    zeros_big = jnp.zeros((N_PAD, H_NF), _F32)
    p0, p1 = _scatter_call(ef, row_pad, zeros_big)
    out8 = _node_mlp_call(nf16, p0, p1, w1n, w1a, n_b1, nW2, n_b2, fw8, fb8)
    return out8[:, :EMB_NF]


# speedup vs baseline: 2.9420x; 1.2203x over previous
import functools

import jax
import jax.numpy as jnp
from jax import lax
from jax.experimental import pallas as pl
from jax.experimental.pallas import tpu as pltpu
from jax.experimental.pallas import tpu_sc as plsc

N_NODES = 10000
N_EDGES = 320000
NODE_NF = 11
EDGE_NF = 4
H_NF = 128
EMB_NF = 4

NC = 2
NS = 16
NW = NC * NS

CH = 128
EPW = 10240
NCH = EPW // CH
E_PAD = EPW * NW
N_PAD = 10112
RPT = N_PAD // NS

_F32 = jnp.float32


def _sc_mesh():
    return plsc.VectorSubcoreMesh(
        core_axis_name="c", subcore_axis_name="s", num_cores=NC, num_subcores=NS
    )


RG = 6
GLEAD = 4


def _gather_call(nf16, row3, col3):
    @functools.partial(
        pl.kernel,
        out_type=(
            jax.ShapeDtypeStruct((E_PAD, 16), _F32),
            jax.ShapeDtypeStruct((E_PAD, 16), _F32),
        ),
        mesh=_sc_mesh(),
        scratch_types=[
            pltpu.VMEM((NCH, CH), jnp.int32),
            pltpu.VMEM((NCH, CH), jnp.int32),
            pltpu.VMEM((RG, CH, 16), _F32),
            pltpu.VMEM((RG, CH, 16), _F32),
            pltpu.SemaphoreType.DMA,
            pltpu.SemaphoreType.DMA,
            pltpu.SemaphoreType.DMA,
            pltpu.SemaphoreType.DMA,
        ],
        compiler_params=pltpu.CompilerParams(use_tc_tiling_on_sc=False),
    )
    def k(nf_hbm, row_hbm, col_hbm, gsrc_hbm, gdst_hbm,
          ridx2, cidx2, sbuf, dbuf, gsem_r, gsem_c, wsem_r, wsem_c):
        wid = lax.axis_index("c") * NS + lax.axis_index("s")

        pltpu.sync_copy(row_hbm.at[wid], ridx2)
        pltpu.sync_copy(col_hbm.at[wid], cidx2)

        def start_gather(t, b):
            pltpu.async_copy(nf_hbm.at[ridx2.at[t]], sbuf.at[b], gsem_r)
            pltpu.async_copy(nf_hbm.at[cidx2.at[t]], dbuf.at[b], gsem_c)

        def wait_gather(t, b):
            pltpu.make_async_copy(nf_hbm.at[ridx2.at[t]], sbuf.at[b], gsem_r).wait()
            pltpu.make_async_copy(nf_hbm.at[cidx2.at[t]], dbuf.at[b], gsem_c).wait()

        def out_slice(hbm, t):
            base = pl.multiple_of(wid * EPW + t * CH, CH)
            return hbm.at[pl.ds(base, CH)]

        def start_write(t, b):
            pltpu.async_copy(sbuf.at[b], out_slice(gsrc_hbm, t), wsem_r)
            pltpu.async_copy(dbuf.at[b], out_slice(gdst_hbm, t), wsem_c)

        def wait_write(t, b):
            pltpu.make_async_copy(sbuf.at[b], out_slice(gsrc_hbm, t), wsem_r).wait()
            pltpu.make_async_copy(dbuf.at[b], out_slice(gdst_hbm, t), wsem_c).wait()

        for t in range(GLEAD):
            start_gather(t, t % RG)

        def body(g, carry):
            for b_off in range(RG):
                t = g * RG + b_off
                b = b_off
                bw = (b_off - 2) % RG

                @pl.when(t >= 2)
                def _():
                    wait_write(t - 2, bw)

                @pl.when(t + GLEAD < NCH)
                def _():
                    start_gather(t + GLEAD, bw)

                wait_gather(t, b)
                start_write(t, b)
            return carry

        lax.fori_loop(0, NCH // RG, body, 0, unroll=False)
        for t in range(NCH - NCH % RG, NCH):
            b = t % RG
            bw = (b - 2) % RG
            wait_write(t - 2, bw)

            @pl.when(t + GLEAD < NCH)
            def _():
                start_gather(t + GLEAD, bw)

            wait_gather(t, b)
            start_write(t, b)
        wait_write(NCH - 2, (NCH - 2) % RG)
        wait_write(NCH - 1, (NCH - 1) % RG)

    return k(nf16, row3, col3)


RS = 2
SLEAD = 1


def _scatter_call(ef, row3, zeros_big):
    @functools.partial(
        pl.kernel,
        out_type=(
            jax.ShapeDtypeStruct((N_PAD, H_NF), _F32),
            jax.ShapeDtypeStruct((N_PAD, H_NF), _F32),
        ),
        mesh=_sc_mesh(),
        scratch_types=[
            pltpu.VMEM((NCH, CH), jnp.int32),
            pltpu.VMEM((RS, CH, H_NF), _F32),
            pltpu.VMEM_SHARED((N_PAD, H_NF), _F32),
            pltpu.SemaphoreType.DMA,
            pltpu.SemaphoreType.DMA,
        ],
        compiler_params=pltpu.CompilerParams(use_tc_tiling_on_sc=False),
    )
    def k(ef_hbm, row_hbm, z_hbm, p0_hbm, p1_hbm, idx2, ebuf, agg_sh,
          lsem, asem):
        c = lax.axis_index("c")
        s = lax.axis_index("s")
        wid = c * NS + s
        rslice = pl.ds(s * RPT, RPT)
        pltpu.sync_copy(z_hbm.at[rslice], agg_sh.at[rslice])
        pltpu.sync_copy(row_hbm.at[wid], idx2)
        plsc.subcore_barrier()

        def ef_slice(t):
            base = pl.multiple_of(wid * EPW + t * CH, CH)
            return ef_hbm.at[pl.ds(base, CH)]

        def start_load(t, b):
            pltpu.async_copy(ef_slice(t), ebuf.at[b], lsem)

        def wait_load(t, b):
            pltpu.make_async_copy(ef_slice(t), ebuf.at[b], lsem).wait()

        def start_add(t, b):
            pltpu.async_copy(ebuf.at[b], agg_sh.at[idx2.at[t]], asem, add=True)

        def wait_add(t, b):
            pltpu.make_async_copy(ebuf.at[b], agg_sh.at[idx2.at[t]], asem).wait()

        start_load(0, 0)

        def body(g, carry):
            for b in range(RS):
                t = g * RS + b
                bo = 1 - b

                @pl.when(t >= 1)
                def _():
                    wait_add(t - 1, bo)

                @pl.when(t + 1 < NCH)
                def _():
                    start_load(t + 1, bo)

                wait_load(t, b)
                start_add(t, b)
            return carry

        lax.fori_loop(0, NCH // RS, body, 0, unroll=False)
        wait_add(NCH - 1, (NCH - 1) % RS)
        plsc.subcore_barrier()

        @pl.when(c == 0)
        def _():
            pltpu.sync_copy(agg_sh.at[rslice], p0_hbm.at[rslice])

        @pl.when(c == 1)
        def _():
            pltpu.sync_copy(agg_sh.at[rslice], p1_hbm.at[rslice])

    return k(ef, row3, zeros_big)


BE = 2048


def _edge_mlp_kernel(gsrc, gdst, ea, w1s, w1d, w1e, b1, w2, b2, out):
    h = jnp.dot(gsrc[...], w1s[...], preferred_element_type=_F32)
    h = h + jnp.dot(gdst[...], w1d[...], preferred_element_type=_F32)
    h = h + jnp.dot(ea[...], w1e[...], preferred_element_type=_F32)
    h = jnp.maximum(h + b1[...], 0.0)
    h = jnp.dot(h, w2[...], preferred_element_type=_F32) + b2[...]
    out[...] = jnp.maximum(h, 0.0)


def _edge_mlp_call(gsrc, gdst, ea8, w1s, w1d, w1e, b1, w2, b2):
    grid = (E_PAD // BE,)
    bcast = lambda shape: pl.BlockSpec(shape, lambda i: (0, 0))
    return pl.pallas_call(
        _edge_mlp_kernel,
        grid=grid,
        in_specs=[
            pl.BlockSpec((BE, 16), lambda i: (i, 0)),
            pl.BlockSpec((BE, 16), lambda i: (i, 0)),
            pl.BlockSpec((BE, 8), lambda i: (i, 0)),
            bcast((16, H_NF)),
            bcast((16, H_NF)),
            bcast((8, H_NF)),
            bcast((1, H_NF)),
            bcast((H_NF, H_NF)),
            bcast((1, H_NF)),
        ],
        out_specs=pl.BlockSpec((BE, H_NF), lambda i: (i, 0)),
        out_shape=jax.ShapeDtypeStruct((E_PAD, H_NF), _F32),
    )(gsrc, gdst, ea8, w1s, w1d, w1e, b1, w2, b2)


BN = 1024


def _node_mlp_kernel(nf, p0, p1, w1n, w1a, b1, w2, b2, fw, fb, out):
    agg = p0[...] + p1[...]
    h = jnp.dot(nf[...], w1n[...], preferred_element_type=_F32)
    h = h + jnp.dot(agg, w1a[...], preferred_element_type=_F32)
    h = jnp.maximum(h + b1[...], 0.0)
    h = jnp.dot(h, w2[...], preferred_element_type=_F32) + b2[...]
    out[...] = jnp.dot(h, fw[...], preferred_element_type=_F32) + fb[...]


def _node_mlp_call(nf16, p0, p1, w1n, w1a, b1, w2, b2, fw8, fb8):
    grid = (pl.cdiv(N_NODES, BN),)
    bcast = lambda shape: pl.BlockSpec(shape, lambda i: (0, 0))
    return pl.pallas_call(
        _node_mlp_kernel,
        grid=grid,
        in_specs=[
            pl.BlockSpec((BN, 16), lambda i: (i, 0)),
            pl.BlockSpec((BN, H_NF), lambda i: (i, 0)),
            pl.BlockSpec((BN, H_NF), lambda i: (i, 0)),
            bcast((16, H_NF)),
            bcast((H_NF, H_NF)),
            bcast((1, H_NF)),
            bcast((H_NF, H_NF)),
            bcast((1, H_NF)),
            bcast((H_NF, 8)),
            bcast((1, 8)),
        ],
        out_specs=pl.BlockSpec((BN, 8), lambda i: (i, 0)),
        out_shape=jax.ShapeDtypeStruct((N_NODES, 8), _F32),
    )(nf16, p0, p1, w1n, w1a, b1, w2, b2, fw8, fb8)


def kernel(node_feats, edge_index, edge_attr,
           eW1, eb1, eW2, eb2, nW1, nb1, nW2, nb2, fW, fb):
    row = edge_index[0]
    col = edge_index[1]
    pad_idx = jnp.full((E_PAD - N_EDGES,), N_NODES, jnp.int32)
    row3 = jnp.concatenate([row, pad_idx]).reshape(NW, NCH, CH)
    col3 = jnp.concatenate([col, pad_idx]).reshape(NW, NCH, CH)

    nf16 = jnp.zeros((N_PAD, 16), _F32).at[:N_NODES, :NODE_NF].set(node_feats)
    ea8 = jnp.zeros((E_PAD, 8), _F32).at[:N_EDGES, :EDGE_NF].set(edge_attr)

    w1s = jnp.zeros((16, H_NF), _F32).at[:NODE_NF].set(eW1[:NODE_NF])
    w1d = jnp.zeros((16, H_NF), _F32).at[:NODE_NF].set(eW1[NODE_NF:2 * NODE_NF])
    w1e = jnp.zeros((8, H_NF), _F32).at[:EDGE_NF].set(eW1[2 * NODE_NF:])
    e_b1 = eb1.reshape(1, H_NF)
    e_b2 = eb2.reshape(1, H_NF)

    w1n = jnp.zeros((16, H_NF), _F32).at[:NODE_NF].set(nW1[:NODE_NF])
    w1a = nW1[NODE_NF:]
    n_b1 = nb1.reshape(1, H_NF)
    n_b2 = nb2.reshape(1, H_NF)
    fw8 = jnp.zeros((H_NF, 8), _F32).at[:, :EMB_NF].set(fW)
    fb8 = jnp.zeros((1, 8), _F32).at[0, :EMB_NF].set(fb)

    gsrc, gdst = _gather_call(nf16, row3, col3)
    ef = _edge_mlp_call(gsrc, gdst, ea8, w1s, w1d, w1e, e_b1, eW2, e_b2)
    zeros_big = jnp.zeros((N_PAD, H_NF), _F32)
    p0, p1 = _scatter_call(ef, row3, zeros_big)
    out8 = _node_mlp_call(nf16, p0, p1, w1n, w1a, n_b1, nW2, n_b2, fw8, fb8)
    return out8[:, :EMB_NF]
